# initial kernel scaffold (unmeasured)
import jax
import jax.numpy as jnp
from jax import lax
from jax.experimental import pallas as pl
from jax.experimental.pallas import tpu as pltpu

N_DEV = 4
B, SQ, DM = 2, 512, 768
HP, DH = 8, 64
HD = HP * DH
SKV = 512
BLK = 64


def kernel(x, Wq, K_ext, V_ext, Wo):
    def body(x_ref, wq_ref, k_ref, v_ref, wo_ref, out_ref,
             kv_buf, stage, ring_buf,
             scat_send_sems, scat_recv_sem, ring_send_sems, ring_recv_sems):
        my = lax.axis_index("i")
        right = (my + 1) % N_DEV

        def scat_rdma(j):
            return pltpu.make_async_remote_copy(
                src_ref=stage.at[j - 1],
                dst_ref=kv_buf,
                send_sem=scat_send_sems.at[j - 1],
                recv_sem=scat_recv_sem,
                device_id=(j,),
                device_id_type=pl.DeviceIdType.MESH,
            )

        @pl.when(my == 0)
        def _():
            for j in range(1, N_DEV):
                stage[j - 1, 0] = k_ref[:, :, j * HP:(j + 1) * HP, :].astype(jnp.bfloat16)
                stage[j - 1, 1] = v_ref[:, :, j * HP:(j + 1) * HP, :].astype(jnp.bfloat16)
                scat_rdma(j).start()
            kv_buf[0] = k_ref[:, :, 0:HP, :].astype(jnp.bfloat16)
            kv_buf[1] = v_ref[:, :, 0:HP, :].astype(jnp.bfloat16)

        x2 = x_ref[...].reshape(B * SQ, DM).astype(jnp.bfloat16)
        wq = wq_ref[...].astype(jnp.bfloat16)
        q2 = jnp.dot(x2, wq, preferred_element_type=jnp.float32)
        q2 = q2.astype(jnp.bfloat16)

        @pl.when(my == 0)
        def _():
            for j in range(1, N_DEV):
                scat_rdma(j).wait_send()

        @pl.when(my != 0)
        def _():
            scat_rdma(1).wait_recv()

        k4 = kv_buf[0]
        v4 = kv_buf[1]
        rb = lax.broadcasted_iota(jnp.int32, (SQ, SKV), 0) // BLK
        cb = lax.broadcasted_iota(jnp.int32, (SQ, SKV), 1) // BLK
        mask = cb <= rb

        row_blocks = []
        for b in range(B):
            cols = []
            for h in range(HP):
                qbh = q2[b * SQ:(b + 1) * SQ, h * DH:(h + 1) * DH]
                kbh = k4[b, :, h, :]
                s = lax.dot_general(
                    qbh, kbh, (((1,), (1,)), ((), ())),
                    preferred_element_type=jnp.float32,
                ) * 0.125
                s = jnp.where(mask, s, -1e9)
                m = jnp.max(s, axis=1, keepdims=True)
                w = jnp.exp(s - m)
                w = w / jnp.sum(w, axis=1, keepdims=True)
                ctx = jnp.dot(w.astype(jnp.bfloat16), v4[b, :, h, :],
                              preferred_element_type=jnp.float32)
                cols.append(ctx.astype(jnp.bfloat16))
            row_blocks.append(jnp.concatenate(cols, axis=1))
        ctx2 = jnp.concatenate(row_blocks, axis=0)

        wo = wo_ref[...].astype(jnp.bfloat16)
        po = jnp.dot(ctx2, wo, preferred_element_type=jnp.float32)
        po = po.reshape(B, SQ, DM)
        out_ref[...] = po
        ring_buf[0] = po.astype(jnp.bfloat16)

        for h in range(N_DEV - 1):
            rdma = pltpu.make_async_remote_copy(
                src_ref=ring_buf.at[h],
                dst_ref=ring_buf.at[h + 1],
                send_sem=ring_send_sems.at[h],
                recv_sem=ring_recv_sems.at[h],
                device_id=(right,),
                device_id_type=pl.DeviceIdType.MESH,
            )
            rdma.start()
            rdma.wait()
            out_ref[...] = out_ref[...] + ring_buf[h + 1].astype(jnp.float32)

    return pl.pallas_call(
        body,
        out_shape=jax.ShapeDtypeStruct((B, SQ, DM), jnp.float32),
        in_specs=[pl.BlockSpec(memory_space=pltpu.VMEM)] * 5,
        out_specs=pl.BlockSpec(memory_space=pltpu.VMEM),
        scratch_shapes=[
            pltpu.VMEM((2, B, SKV, HP, DH), jnp.bfloat16),
            pltpu.VMEM((3, 2, B, SKV, HP, DH), jnp.bfloat16),
            pltpu.VMEM((N_DEV, B, SQ, DM), jnp.bfloat16),
            pltpu.SemaphoreType.DMA((3,)),
            pltpu.SemaphoreType.DMA,
            pltpu.SemaphoreType.DMA((3,)),
            pltpu.SemaphoreType.DMA((3,)),
        ],
    )(x, Wq, K_ext, V_ext, Wo)


# baseline (device time: 215271 ns/iter reference)
import jax
import jax.numpy as jnp
from jax import lax
from jax.experimental import pallas as pl
from jax.experimental.pallas import tpu as pltpu

N_DEV = 4
B, SQ, DM = 2, 512, 768
HP, DH = 8, 64
HD = HP * DH
SKV = 512
BLK = 64


def kernel(x, Wq, K_ext, V_ext, Wo):
    def body(x_ref, wq_ref, k_ref, v_ref, wo_ref, out_ref,
             kv_buf, stage, kvf32, ring_buf,
             scat_send_sems, scat_recv_sem, ring_send_sems, ring_recv_sems,
             local_sems):
        my = lax.axis_index("i")
        right = (my + 1) % N_DEV

        def scat_rdma(j):
            return pltpu.make_async_remote_copy(
                src_ref=stage.at[j - 1],
                dst_ref=kv_buf,
                send_sem=scat_send_sems.at[j - 1],
                recv_sem=scat_recv_sem,
                device_id=(j,),
                device_id_type=pl.DeviceIdType.MESH,
            )

        @pl.when(my == 0)
        def _():
            for j in range(N_DEV):
                hs, he = j * HP, (j + 1) * HP
                ck = pltpu.make_async_copy(
                    k_ref.at[:, :, hs:he, :], kvf32.at[0], local_sems.at[0])
                cv = pltpu.make_async_copy(
                    v_ref.at[:, :, hs:he, :], kvf32.at[1], local_sems.at[1])
                ck.start()
                cv.start()
                ck.wait()
                cv.wait()
                if j == 0:
                    kv_buf[0] = kvf32[0].astype(jnp.bfloat16)
                    kv_buf[1] = kvf32[1].astype(jnp.bfloat16)
                else:
                    stage[j - 1, 0] = kvf32[0].astype(jnp.bfloat16)
                    stage[j - 1, 1] = kvf32[1].astype(jnp.bfloat16)
                    scat_rdma(j).start()

        x2 = x_ref[...].reshape(B * SQ, DM).astype(jnp.bfloat16)
        wq = wq_ref[...].astype(jnp.bfloat16)
        q2 = jnp.dot(x2, wq, preferred_element_type=jnp.float32)
        q2 = q2.astype(jnp.bfloat16)

        @pl.when(my == 0)
        def _():
            for j in range(1, N_DEV):
                scat_rdma(j).wait_send()

        @pl.when(my != 0)
        def _():
            scat_rdma(1).wait_recv()

        k4 = kv_buf[0]
        v4 = kv_buf[1]
        rb = lax.broadcasted_iota(jnp.int32, (SQ, SKV), 0) // BLK
        cb = lax.broadcasted_iota(jnp.int32, (SQ, SKV), 1) // BLK
        mask = cb <= rb

        row_blocks = []
        for b in range(B):
            cols = []
            for h in range(HP):
                qbh = q2[b * SQ:(b + 1) * SQ, h * DH:(h + 1) * DH]
                kbh = k4[b, :, h, :]
                s = lax.dot_general(
                    qbh, kbh, (((1,), (1,)), ((), ())),
                    preferred_element_type=jnp.float32,
                ) * 0.125
                s = jnp.where(mask, s, -1e9)
                m = jnp.max(s, axis=1, keepdims=True)
                w = jnp.exp(s - m)
                w = w / jnp.sum(w, axis=1, keepdims=True)
                ctx = jnp.dot(w.astype(jnp.bfloat16), v4[b, :, h, :],
                              preferred_element_type=jnp.float32)
                cols.append(ctx.astype(jnp.bfloat16))
            row_blocks.append(jnp.concatenate(cols, axis=1))
        ctx2 = jnp.concatenate(row_blocks, axis=0)

        wo = wo_ref[...].astype(jnp.bfloat16)
        po = jnp.dot(ctx2, wo, preferred_element_type=jnp.float32)
        po = po.reshape(B, SQ, DM)
        out_ref[...] = po
        ring_buf[0] = po.astype(jnp.bfloat16)

        for h in range(N_DEV - 1):
            rdma = pltpu.make_async_remote_copy(
                src_ref=ring_buf.at[h],
                dst_ref=ring_buf.at[h + 1],
                send_sem=ring_send_sems.at[h],
                recv_sem=ring_recv_sems.at[h],
                device_id=(right,),
                device_id_type=pl.DeviceIdType.MESH,
            )
            rdma.start()
            rdma.wait()
            out_ref[...] = out_ref[...] + ring_buf[h + 1].astype(jnp.float32)

    return pl.pallas_call(
        body,
        out_shape=jax.ShapeDtypeStruct((B, SQ, DM), jnp.float32),
        in_specs=[
            pl.BlockSpec(memory_space=pltpu.VMEM),
            pl.BlockSpec(memory_space=pltpu.VMEM),
            pl.BlockSpec(memory_space=pl.ANY),
            pl.BlockSpec(memory_space=pl.ANY),
            pl.BlockSpec(memory_space=pltpu.VMEM),
        ],
        out_specs=pl.BlockSpec(memory_space=pltpu.VMEM),
        scratch_shapes=[
            pltpu.VMEM((2, B, SKV, HP, DH), jnp.bfloat16),
            pltpu.VMEM((3, 2, B, SKV, HP, DH), jnp.bfloat16),
            pltpu.VMEM((2, B, SKV, HP, DH), jnp.float32),
            pltpu.VMEM((N_DEV, B, SQ, DM), jnp.bfloat16),
            pltpu.SemaphoreType.DMA((3,)),
            pltpu.SemaphoreType.DMA,
            pltpu.SemaphoreType.DMA((3,)),
            pltpu.SemaphoreType.DMA((3,)),
            pltpu.SemaphoreType.DMA((2,)),
        ],
        compiler_params=pltpu.CompilerParams(
            vmem_limit_bytes=64 * 1024 * 1024,
        ),
    )(x, Wq, K_ext, V_ext, Wo)


# device time: 141917 ns/iter; 1.5169x vs baseline; 1.5169x over previous
import jax
import jax.numpy as jnp
from jax import lax
from jax.experimental import pallas as pl
from jax.experimental.pallas import tpu as pltpu

N_DEV = 4
B, SQ, DM = 2, 512, 768
HP, DH = 8, 64
HD = HP * DH
SKV = 512
BLK = 64
RQ = B * SQ
CH = RQ // N_DEV


def kernel(x, Wq, K_ext, V_ext, Wo):
    K2 = K_ext.astype(jnp.bfloat16).reshape(RQ, N_DEV * HD)
    V2 = V_ext.astype(jnp.bfloat16).reshape(RQ, N_DEV * HD)

    def body(x_ref, wq_ref, k_ref, v_ref, wo_ref, out_ref,
             kv_buf, pbuf, send0_buf, rs_sbuf, rs_rbuf, ag_buf,
             scat_send_sems, scat_recv_sems,
             rs_send_sems, rs_recv_sems, ag_send_sems, ag_recv_sems,
             local_sems):
        my = lax.axis_index("i")
        right = (my + 1) % N_DEV

        def kv_rdma(kv, j):
            src = k_ref if kv == 0 else v_ref
            return pltpu.make_async_remote_copy(
                src_ref=src.at[:, j * HD:(j + 1) * HD],
                dst_ref=kv_buf.at[kv],
                send_sem=scat_send_sems.at[3 * kv + j - 1],
                recv_sem=scat_recv_sems.at[kv],
                device_id=(j,),
                device_id_type=pl.DeviceIdType.MESH,
            )

        def local_kv_copy(kv):
            src = k_ref if kv == 0 else v_ref
            return pltpu.make_async_copy(
                src.at[:, 0:HD], kv_buf.at[kv], local_sems.at[kv])

        barrier = pltpu.get_barrier_semaphore()

        @pl.when(my != 0)
        def _():
            pl.semaphore_signal(barrier, inc=1, device_id=(0,),
                                device_id_type=pl.DeviceIdType.MESH)

        @pl.when(my == 0)
        def _():
            pl.semaphore_wait(barrier, N_DEV - 1)
            for j in range(1, N_DEV):
                kv_rdma(0, j).start()
            local_kv_copy(0).start()
            local_kv_copy(1).start()

        x2 = x_ref[...].reshape(RQ, DM).astype(jnp.bfloat16)
        q2 = jnp.dot(x2, wq_ref[...].astype(jnp.bfloat16),
                     preferred_element_type=jnp.float32).astype(jnp.bfloat16)

        @pl.when(my == 0)
        def _():
            for j in range(1, N_DEV):
                kv_rdma(0, j).wait_send()
            for j in range(1, N_DEV):
                kv_rdma(1, j).start()
            local_kv_copy(0).wait()

        @pl.when(my != 0)
        def _():
            kv_rdma(0, 1).wait_recv()

        rb = lax.broadcasted_iota(jnp.int32, (SQ, SKV), 0) // BLK
        cb = lax.broadcasted_iota(jnp.int32, (SQ, SKV), 1) // BLK
        madd = jnp.where(cb <= rb, 0.0, -1e9).astype(jnp.float32)

        kslab = kv_buf[0]
        ws = []
        for b in range(B):
            for h in range(HP):
                qbh = q2[b * SQ:(b + 1) * SQ, h * DH:(h + 1) * DH]
                kbh = kslab[b * SQ:(b + 1) * SQ, h * DH:(h + 1) * DH]
                s = lax.dot_general(
                    qbh, kbh, (((1,), (1,)), ((), ())),
                    preferred_element_type=jnp.float32,
                ) * 0.125 + madd
                m = jnp.max(s, axis=1, keepdims=True)
                e = jnp.exp(s - m)
                w = e / jnp.sum(e, axis=1, keepdims=True)
                ws.append(w.astype(jnp.bfloat16))

        @pl.when(my == 0)
        def _():
            local_kv_copy(1).wait()

        @pl.when(my != 0)
        def _():
            kv_rdma(1, 1).wait_recv()

        vslab = kv_buf[1]
        row_blocks = []
        for b in range(B):
            cols = []
            for h in range(HP):
                vbh = vslab[b * SQ:(b + 1) * SQ, h * DH:(h + 1) * DH]
                ctx = jnp.dot(ws[b * HP + h], vbh,
                              preferred_element_type=jnp.float32)
                cols.append(ctx.astype(jnp.bfloat16))
            row_blocks.append(jnp.concatenate(cols, axis=1))
        ctx2 = jnp.concatenate(row_blocks, axis=0)
        po2 = jnp.dot(ctx2, wo_ref[...].astype(jnp.bfloat16),
                      preferred_element_type=jnp.float32)

        @pl.when(my == 0)
        def _():
            for j in range(1, N_DEV):
                kv_rdma(1, j).wait_send()

        pbuf[...] = po2

        def chunk(c):
            return pbuf[pl.ds(c * CH, CH), :]

        def ring_send(src_ref, dst_ref, ssem, rsem):
            return pltpu.make_async_remote_copy(
                src_ref=src_ref, dst_ref=dst_ref,
                send_sem=ssem, recv_sem=rsem,
                device_id=(right,), device_id_type=pl.DeviceIdType.MESH)

        send0_buf[...] = chunk(my).astype(jnp.bfloat16)
        acc = None
        rs_rdmas = []
        for s in range(N_DEV - 1):
            src = send0_buf if s == 0 else rs_sbuf.at[s - 1]
            rdma = ring_send(src, rs_rbuf.at[s],
                             rs_send_sems.at[s], rs_recv_sems.at[s])
            rs_rdmas.append(rdma)
            rdma.start()
            rdma.wait_recv()
            acc = chunk((my + N_DEV - 1 - s) % N_DEV) \
                + rs_rbuf[s].astype(jnp.float32)
            if s < N_DEV - 2:
                rs_sbuf[s] = acc.astype(jnp.bfloat16)

        red_idx = (my + 1) % N_DEV
        ag_buf[0] = acc.astype(jnp.bfloat16)
        out_ref[pl.ds(red_idx * CH, CH), :] = acc

        ag_rdmas = [
            ring_send(ag_buf.at[t], ag_buf.at[t + 1],
                      ag_send_sems.at[t], ag_recv_sems.at[t])
            for t in range(N_DEV - 1)
        ]
        ag_rdmas[0].start()
        for t in range(N_DEV - 1):
            ag_rdmas[t].wait_recv()
            if t + 1 < N_DEV - 1:
                ag_rdmas[t + 1].start()
            cidx = (my + N_DEV - t) % N_DEV
            out_ref[pl.ds(cidx * CH, CH), :] = ag_buf[t + 1].astype(jnp.float32)

        for r in rs_rdmas + ag_rdmas:
            r.wait_send()

    out2 = pl.pallas_call(
        body,
        out_shape=jax.ShapeDtypeStruct((RQ, DM), jnp.float32),
        in_specs=[
            pl.BlockSpec(memory_space=pltpu.VMEM),
            pl.BlockSpec(memory_space=pltpu.VMEM),
            pl.BlockSpec(memory_space=pl.ANY),
            pl.BlockSpec(memory_space=pl.ANY),
            pl.BlockSpec(memory_space=pltpu.VMEM),
        ],
        out_specs=pl.BlockSpec(memory_space=pltpu.VMEM),
        scratch_shapes=[
            pltpu.VMEM((2, RQ, HD), jnp.bfloat16),
            pltpu.VMEM((RQ, DM), jnp.float32),
            pltpu.VMEM((CH, DM), jnp.bfloat16),
            pltpu.VMEM((2, CH, DM), jnp.bfloat16),
            pltpu.VMEM((3, CH, DM), jnp.bfloat16),
            pltpu.VMEM((N_DEV, CH, DM), jnp.bfloat16),
            pltpu.SemaphoreType.DMA((6,)),
            pltpu.SemaphoreType.DMA((2,)),
            pltpu.SemaphoreType.DMA((3,)),
            pltpu.SemaphoreType.DMA((3,)),
            pltpu.SemaphoreType.DMA((3,)),
            pltpu.SemaphoreType.DMA((3,)),
            pltpu.SemaphoreType.DMA((2,)),
        ],
        compiler_params=pltpu.CompilerParams(
            collective_id=0,
            vmem_limit_bytes=64 * 1024 * 1024,
        ),
    )(x, Wq, K2, V2, Wo)
    return out2.reshape(B, SQ, DM)
